# baseline (speedup 1.0000x reference)
import functools

import jax
import jax.numpy as jnp
from jax import lax
from jax.experimental import pallas as pl
from jax.experimental.pallas import tpu as pltpu
from jax.experimental.pallas import tpu_sc as plsc

BATCH = 4
SEQ_LEN = 4096
DIM = 2048

NC, NS = 2, 16
NW = NC * NS
SPW = SEQ_LEN // NW
R = 2
NBUF = 6
PD = 3
HEAD = max(PD, NBUF - PD)
NSTEP = SPW // R

_mesh = plsc.VectorSubcoreMesh(
    core_axis_name="c", subcore_axis_name="s", num_cores=NC, num_subcores=NS
)

_scratch = (
    [pltpu.VMEM((R, DIM), jnp.float32) for _ in range(NBUF)]
    + [pltpu.VMEM((R, DIM), jnp.float32) for _ in range(NBUF)]
    + [pltpu.SemaphoreType.DMA for _ in range(3 * NBUF)]
)


@functools.partial(
    pl.kernel,
    out_type=jax.ShapeDtypeStruct((1, SEQ_LEN, DIM), jnp.float32),
    mesh=_mesh,
    scratch_types=_scratch,
)
def _sc_add(x_hbm, e_hbm, o_hbm, *scr):
    xb = scr[0:NBUF]
    eb = scr[NBUF : 2 * NBUF]
    gx = scr[2 * NBUF : 3 * NBUF]
    ge = scr[3 * NBUF : 4 * NBUF]
    st = scr[4 * NBUF : 5 * NBUF]

    wid = lax.axis_index("s") * NC + lax.axis_index("c")
    t0 = wid * SPW

    def gather(j, s):
        pltpu.async_copy(e_hbm.at[pl.ds(t0 + j * R, R)], eb[s], ge[s])
        pltpu.async_copy(x_hbm.at[0, pl.ds(t0 + j * R, R), :], xb[s], gx[s])

    def gather_wait(j, s):
        pltpu.make_async_copy(e_hbm.at[pl.ds(t0 + j * R, R)], eb[s], ge[s]).wait()
        pltpu.make_async_copy(
            x_hbm.at[0, pl.ds(t0 + j * R, R), :], xb[s], gx[s]
        ).wait()

    def store(j, s):
        pltpu.async_copy(xb[s], o_hbm.at[0, pl.ds(t0 + j * R, R), :], st[s])

    def store_wait(j, s):
        pltpu.make_async_copy(
            xb[s], o_hbm.at[0, pl.ds(t0 + j * R, R), :], st[s]
        ).wait()

    def compute(s):
        for r in range(R):
            @pl.loop(0, DIM // 16, unroll=8)
            def _col(c):
                cs = pl.ds(c * 16, 16)
                plsc.addupdate(xb[s].at[r, cs], eb[s][r, cs])

    def step(j, s):
        gather_wait(j, s)
        compute(s)
        store(j, s)
        tgt = j + PD
        ts = (s + PD) % NBUF
        if isinstance(tgt, int):
            if tgt < NSTEP:
                if tgt - NBUF >= 0:
                    store_wait(tgt - NBUF, ts)
                gather(tgt, ts)
        else:
            store_wait(tgt - NBUF, ts)
            gather(tgt, ts)

    for j in range(PD):
        gather(j, j)
    for j in range(HEAD):
        step(j, j % NBUF)

    G = (NSTEP - HEAD - PD) // NBUF

    @pl.loop(0, G)
    def _block(g):
        for k in range(NBUF):
            s = (HEAD + k) % NBUF
            step(HEAD + g * NBUF + k, s)

    for j in range(HEAD + G * NBUF, NSTEP):
        step(j, j % NBUF)
    for j in range(NSTEP - NBUF, NSTEP):
        store_wait(j, j % NBUF)


BS = 512


def _tc_body(x_ref, e_ref, o_ref):
    o_ref[...] = x_ref[...] + e_ref[...][None]


def _tc_add(x, embed):
    grid = (SEQ_LEN // BS, BATCH - 1)
    return pl.pallas_call(
        _tc_body,
        grid=grid,
        in_specs=[
            pl.BlockSpec((1, BS, DIM), lambda s, b: (b + 1, s, 0)),
            pl.BlockSpec((BS, DIM), lambda s, b: (s, 0)),
        ],
        out_specs=pl.BlockSpec((1, BS, DIM), lambda s, b: (b, s, 0)),
        out_shape=jax.ShapeDtypeStruct((BATCH - 1, SEQ_LEN, DIM), jnp.float32),
    )(x, embed)


def kernel(x, embed):
    sc_out = _sc_add(x, embed)
    tc_out = _tc_add(x, embed)
    return jnp.concatenate([sc_out, tc_out], axis=0)


# speedup vs baseline: 1.5638x; 1.5638x over previous
import functools

import jax
import jax.numpy as jnp
from jax import lax
from jax.experimental import pallas as pl
from jax.experimental.pallas import tpu as pltpu
from jax.experimental.pallas import tpu_sc as plsc

BATCH = 4
SEQ_LEN = 4096
DIM = 2048

NC, NS = 2, 16
NW = NC * NS
SPW = SEQ_LEN // NW
R = 2
NBUF = 6
PD = 3
HEAD = max(PD, NBUF - PD)
NSTEP = SPW // R

_mesh = plsc.VectorSubcoreMesh(
    core_axis_name="c", subcore_axis_name="s", num_cores=NC, num_subcores=NS
)

_scratch = (
    [pltpu.VMEM((BATCH, R, DIM), jnp.float32) for _ in range(NBUF)]
    + [pltpu.VMEM((R, DIM), jnp.float32) for _ in range(NBUF)]
    + [pltpu.SemaphoreType.DMA for _ in range(3 * NBUF)]
)


@functools.partial(
    pl.kernel,
    out_type=jax.ShapeDtypeStruct((BATCH, SEQ_LEN, DIM), jnp.float32),
    mesh=_mesh,
    scratch_types=_scratch,
)
def _sc_add(x_hbm, e_hbm, o_hbm, *scr):
    xb = scr[0:NBUF]
    eb = scr[NBUF : 2 * NBUF]
    gx = scr[2 * NBUF : 3 * NBUF]
    ge = scr[3 * NBUF : 4 * NBUF]
    st = scr[4 * NBUF : 5 * NBUF]

    wid = lax.axis_index("s") * NC + lax.axis_index("c")
    t0 = wid * SPW

    def gather(j, s):
        pltpu.async_copy(e_hbm.at[pl.ds(t0 + j * R, R)], eb[s], ge[s])
        pltpu.async_copy(x_hbm.at[:, pl.ds(t0 + j * R, R), :], xb[s], gx[s])

    def gather_wait(j, s):
        pltpu.make_async_copy(e_hbm.at[pl.ds(t0 + j * R, R)], eb[s], ge[s]).wait()
        pltpu.make_async_copy(
            x_hbm.at[:, pl.ds(t0 + j * R, R), :], xb[s], gx[s]
        ).wait()

    def store_row(j, s, r):
        pltpu.async_copy(
            xb[s].at[:, pl.ds(r, 1), :],
            o_hbm.at[:, pl.ds(t0 + j * R + r, 1), :],
            st[s],
        )

    def store_wait(j, s):
        for r in range(R):
            pltpu.make_async_copy(
                xb[s].at[:, pl.ds(r, 1), :],
                o_hbm.at[:, pl.ds(t0 + j * R + r, 1), :],
                st[s],
            ).wait()

    def compute_row(s, r):
        @pl.loop(0, DIM // 16, unroll=8)
        def _col(c):
            cs = pl.ds(c * 16, 16)
            ev = eb[s][r, cs]
            for b in range(BATCH):
                plsc.addupdate(xb[s].at[b, r, cs], ev)

    def step(j, s):
        gather_wait(j, s)
        for r in range(R):
            compute_row(s, r)
            store_row(j, s, r)
        tgt = j + PD
        ts = (s + PD) % NBUF
        if isinstance(tgt, int):
            if tgt < NSTEP:
                if tgt - NBUF >= 0:
                    store_wait(tgt - NBUF, ts)
                gather(tgt, ts)
        else:
            store_wait(tgt - NBUF, ts)
            gather(tgt, ts)

    for j in range(PD):
        gather(j, j)

    for j in range(HEAD):
        step(j, j % NBUF)

    G = (NSTEP - HEAD - PD) // NBUF

    @pl.loop(0, G)
    def _block(g):
        for k in range(NBUF):
            s = (HEAD + k) % NBUF
            step(HEAD + g * NBUF + k, s)

    for j in range(HEAD + G * NBUF, NSTEP):
        step(j, j % NBUF)

    for j in range(NSTEP - NBUF, NSTEP):
        store_wait(j, j % NBUF)


def kernel(x, embed):
    return _sc_add(x, embed)


# speedup vs baseline: 1.6310x; 1.0430x over previous
import functools

import jax
import jax.numpy as jnp
from jax import lax
from jax.experimental import pallas as pl
from jax.experimental.pallas import tpu as pltpu
from jax.experimental.pallas import tpu_sc as plsc

BATCH = 4
SEQ_LEN = 4096
DIM = 2048

NC, NS = 2, 16
NW = NC * NS
SPW = SEQ_LEN // NW
R = 2
NBUF = 4
PD = 2
HEAD = max(PD, NBUF - PD)
NSTEP = SPW // R

_mesh = plsc.VectorSubcoreMesh(
    core_axis_name="c", subcore_axis_name="s", num_cores=NC, num_subcores=NS
)

_scratch = (
    [pltpu.VMEM((BATCH, R, DIM), jnp.float32) for _ in range(NBUF)]
    + [pltpu.VMEM((R, DIM), jnp.float32) for _ in range(NBUF)]
    + [pltpu.SemaphoreType.DMA for _ in range(3 * NBUF)]
)


@functools.partial(
    pl.kernel,
    out_type=jax.ShapeDtypeStruct((BATCH, SEQ_LEN, DIM), jnp.float32),
    mesh=_mesh,
    scratch_types=_scratch,
)
def _sc_add(x_hbm, e_hbm, o_hbm, *scr):
    xb = scr[0:NBUF]
    eb = scr[NBUF : 2 * NBUF]
    gx = scr[2 * NBUF : 3 * NBUF]
    ge = scr[3 * NBUF : 4 * NBUF]
    st = scr[4 * NBUF : 5 * NBUF]

    wid = lax.axis_index("s") * NC + lax.axis_index("c")
    t0 = wid * SPW

    def gather(j, s):
        pltpu.async_copy(e_hbm.at[pl.ds(t0 + j * R, R)], eb[s], ge[s])
        pltpu.async_copy(x_hbm.at[:, pl.ds(t0 + j * R, R), :], xb[s], gx[s])

    def gather_wait(j, s):
        pltpu.make_async_copy(e_hbm.at[pl.ds(t0 + j * R, R)], eb[s], ge[s]).wait()
        pltpu.make_async_copy(
            x_hbm.at[:, pl.ds(t0 + j * R, R), :], xb[s], gx[s]
        ).wait()

    def store(j, s):
        pltpu.async_copy(xb[s], o_hbm.at[:, pl.ds(t0 + j * R, R), :], st[s])

    def store_wait(j, s):
        pltpu.make_async_copy(
            xb[s], o_hbm.at[:, pl.ds(t0 + j * R, R), :], st[s]
        ).wait()

    def compute(s):
        for r in range(R):
            @pl.loop(0, DIM // 16, unroll=8)
            def _col(c):
                cs = pl.ds(c * 16, 16)
                ev = eb[s][r, cs]
                for b in range(BATCH):
                    plsc.addupdate(xb[s].at[b, r, cs], ev)

    def step(j, s):
        gather_wait(j, s)
        compute(s)
        store(j, s)
        tgt = j + PD
        ts = (s + PD) % NBUF
        if isinstance(tgt, int):
            if tgt < NSTEP:
                if tgt - NBUF >= 0:
                    store_wait(tgt - NBUF, ts)
                gather(tgt, ts)
        else:
            store_wait(tgt - NBUF, ts)
            gather(tgt, ts)

    for j in range(PD):
        gather(j, j)

    for j in range(HEAD):
        step(j, j % NBUF)

    G = (NSTEP - HEAD - PD) // NBUF

    @pl.loop(0, G)
    def _block(g):
        for k in range(NBUF):
            s = (HEAD + k) % NBUF
            step(HEAD + g * NBUF + k, s)

    for j in range(HEAD + G * NBUF, NSTEP):
        step(j, j % NBUF)

    for j in range(NSTEP - NBUF, NSTEP):
        store_wait(j, j % NBUF)


def kernel(x, embed):
    return _sc_add(x, embed)
